# baseline (device time: 1500151 ns/iter reference)
import jax
import jax.numpy as jnp
from jax import lax
from jax.experimental import pallas as pl
from jax.experimental.pallas import tpu as pltpu

N_DEV = 16
M = 4096
N = 8192
HALF = M // 2
CHUNK = HALF // N_DEV
SUBS = 4
SUB = CHUNK // SUBS
N_STEPS = 2 * (N_DEV - 1)
MESH = pl.DeviceIdType.MESH


def _allreduce(partial):
    def body(p_ref, out_ref,
             acc_r, recv_r, local_r, acc_l, recv_l, local_l,
             send_sems_r, recv_sems_r, copy_sems_r, store_sems_r,
             send_sems_l, recv_sems_l, copy_sems_l, store_sems_l,
             credit_r, credit_l):
        my = lax.axis_index("i")
        left = jnp.mod(my - 1, N_DEV)
        right = jnp.mod(my + 1, N_DEV)

        dirs = (
            dict(peer=right, ack=left, base=0, acc=acc_r, recv=recv_r,
                 local=local_r, send_sems=send_sems_r, recv_sems=recv_sems_r,
                 copy_sems=copy_sems_r, store_sems=store_sems_r,
                 credit=credit_r, sign=-1),
            dict(peer=left, ack=right, base=HALF, acc=acc_l, recv=recv_l,
                 local=local_l, send_sems=send_sems_l, recv_sems=recv_sems_l,
                 copy_sems=copy_sems_l, store_sems=store_sems_l,
                 credit=credit_l, sign=+1),
        )

        def rdma(d, t, j):
            p = t % 2
            return pltpu.make_async_remote_copy(
                src_ref=d["acc"].at[p, pl.ds(j * SUB, SUB), :],
                dst_ref=d["recv"].at[p, pl.ds(j * SUB, SUB), :],
                send_sem=d["send_sems"].at[p, j],
                recv_sem=d["recv_sems"].at[p, j],
                device_id=(d["peer"],), device_id_type=MESH)

        def in_idx(d, t):
            return jnp.mod(my + d["sign"] * (t + 1), N_DEV)

        def load(d, t, j):
            p = t % 2
            rows = in_idx(d, t) * CHUNK + j * SUB
            return pltpu.make_async_copy(
                p_ref.at[pl.ds(d["base"] + rows, SUB), :],
                d["local"].at[p, pl.ds(j * SUB, SUB), :],
                d["copy_sems"].at[p, j])

        def store(d, t):
            q = (t + 1) % 2
            if t == N_DEV - 2:
                idx = jnp.mod(my - d["sign"], N_DEV)
            else:
                idx = jnp.mod(my + d["sign"] * (t - (N_DEV - 1)), N_DEV)
            return pltpu.make_async_copy(
                d["acc"].at[q],
                out_ref.at[pl.ds(d["base"] + idx * CHUNK, CHUNK), :],
                d["store_sems"].at[q])

        barrier = pltpu.get_barrier_semaphore()
        for nbr in (left, right):
            pl.semaphore_signal(barrier, inc=1, device_id=(nbr,),
                                device_id_type=MESH)
        pl.semaphore_wait(barrier, 2)

        for d in dirs:
            cp = pltpu.make_async_copy(
                p_ref.at[pl.ds(d["base"] + my * CHUNK, CHUNK), :],
                d["acc"].at[0], d["copy_sems"].at[0, 0])
            cp.start()
            cp.wait()
        for d in dirs:
            for j in range(SUBS):
                rdma(d, 0, j).start()
                load(d, 0, j).start()

        pending_stores = {0: [], 1: []}
        for t in range(N_STEPS):
            p = t % 2
            np_ = (t + 1) % 2
            rs = t < N_DEV - 1

            if t + 1 < N_DEV - 1:
                for d in dirs:
                    for j in range(SUBS):
                        load(d, t + 1, j).start()

            for st in pending_stores[np_]:
                st.wait()
            pending_stores[np_] = []

            for j in range(SUBS):
                for d in dirs:
                    rdma(d, t, j).wait_recv()
                    if t >= 1:
                        rdma(d, t - 1, j).wait_send()
                    sl = slice(j * SUB, (j + 1) * SUB)
                    if rs:
                        load(d, t, j).wait()
                        d["acc"][np_, sl, :] = (
                            d["recv"][p, sl, :] + d["local"][p, sl, :])
                    else:
                        d["acc"][np_, sl, :] = d["recv"][p, sl, :]
                    if t <= N_STEPS - 3:
                        pl.semaphore_signal(d["credit"], inc=1,
                                            device_id=(d["ack"],),
                                            device_id_type=MESH)
                    if t + 1 < N_STEPS:
                        if t + 1 >= 2:
                            pl.semaphore_wait(d["credit"], 1)
                        rdma(d, t + 1, j).start()

            if t >= N_DEV - 2:
                for d in dirs:
                    st = store(d, t)
                    st.start()
                    pending_stores[np_].append(st)

        for d in dirs:
            for j in range(SUBS):
                rdma(d, N_STEPS - 1, j).wait_send()
        for q in (0, 1):
            for st in pending_stores[q]:
                st.wait()

    return pl.pallas_call(
        body,
        out_shape=jax.ShapeDtypeStruct((M, N), jnp.float32),
        in_specs=[pl.BlockSpec(memory_space=pl.ANY)],
        out_specs=pl.BlockSpec(memory_space=pl.ANY),
        scratch_shapes=[
            pltpu.VMEM((2, CHUNK, N), jnp.float32),
            pltpu.VMEM((2, CHUNK, N), jnp.float32),
            pltpu.VMEM((2, CHUNK, N), jnp.float32),
            pltpu.VMEM((2, CHUNK, N), jnp.float32),
            pltpu.VMEM((2, CHUNK, N), jnp.float32),
            pltpu.VMEM((2, CHUNK, N), jnp.float32),
            pltpu.SemaphoreType.DMA((2, SUBS)),
            pltpu.SemaphoreType.DMA((2, SUBS)),
            pltpu.SemaphoreType.DMA((2, SUBS)),
            pltpu.SemaphoreType.DMA((2,)),
            pltpu.SemaphoreType.DMA((2, SUBS)),
            pltpu.SemaphoreType.DMA((2, SUBS)),
            pltpu.SemaphoreType.DMA((2, SUBS)),
            pltpu.SemaphoreType.DMA((2,)),
            pltpu.SemaphoreType.REGULAR,
            pltpu.SemaphoreType.REGULAR,
        ],
        compiler_params=pltpu.CompilerParams(
            collective_id=0, vmem_limit_bytes=80 * 1024 * 1024),
    )(partial)


def kernel(x, w_mat, scale_x, scale_w):
    s = scale_x[0].astype(jnp.float32) * scale_w[0].astype(jnp.float32)
    partial = lax.dot_general(
        x, w_mat, (((1,), (0,)), ((), ())),
        preferred_element_type=jnp.float32)
    partial = partial * s
    return _allreduce(partial)


# device time: 825360 ns/iter; 1.8176x vs baseline; 1.8176x over previous
import jax
import jax.numpy as jnp
from jax import lax
from jax.experimental import pallas as pl
from jax.experimental.pallas import tpu as pltpu

N_DEV = 16
M = 4096
N = 8192
HALF = M // 2
CHUNK = HALF // N_DEV
SUBS = 2
SUB = CHUNK // SUBS
N_STEPS = 2 * (N_DEV - 1)
MESH = pl.DeviceIdType.MESH


def _allreduce(partial):
    def body(p_ref, out_ref,
             send_r, recv_r, local_r, fst_r, send_l, recv_l, local_l, fst_l,
             send_sems_r, recv_sems_r, copy_sems_r, store_sems_r,
             send_sems_l, recv_sems_l, copy_sems_l, store_sems_l,
             credit_r, credit_l):
        my = lax.axis_index("i")
        left = jnp.mod(my - 1, N_DEV)
        right = jnp.mod(my + 1, N_DEV)

        dirs = (
            dict(peer=right, ack=left, base=0, send=send_r, recv=recv_r,
                 local=local_r, fst=fst_r, send_sems=send_sems_r,
                 recv_sems=recv_sems_r, copy_sems=copy_sems_r,
                 store_sems=store_sems_r, credit=credit_r, sign=-1),
            dict(peer=left, ack=right, base=HALF, send=send_l, recv=recv_l,
                 local=local_l, fst=fst_l, send_sems=send_sems_l,
                 recv_sems=recv_sems_l, copy_sems=copy_sems_l,
                 store_sems=store_sems_l, credit=credit_l, sign=+1),
        )

        def rdma(d, t, j):
            p = t % 2
            return pltpu.make_async_remote_copy(
                src_ref=d["send"].at[p, pl.ds(j * SUB, SUB), :],
                dst_ref=d["recv"].at[p, pl.ds(j * SUB, SUB), :],
                send_sem=d["send_sems"].at[p, j],
                recv_sem=d["recv_sems"].at[p, j],
                device_id=(d["peer"],), device_id_type=MESH)

        def in_idx(d, t):
            return jnp.mod(my + d["sign"] * (t + 1), N_DEV)

        def load(d, t, j):
            p = t % 2
            rows = in_idx(d, t) * CHUNK + j * SUB
            return pltpu.make_async_copy(
                p_ref.at[pl.ds(d["base"] + rows, SUB), :],
                d["local"].at[p, pl.ds(j * SUB, SUB), :],
                d["copy_sems"].at[p, j])

        def store(d, t):
            q = (t + 1) % 2
            if t == N_DEV - 2:
                idx = jnp.mod(my - d["sign"], N_DEV)
            else:
                idx = jnp.mod(my + d["sign"] * (t - (N_DEV - 1)), N_DEV)
            return pltpu.make_async_copy(
                d["fst"].at[q],
                out_ref.at[pl.ds(d["base"] + idx * CHUNK, CHUNK), :],
                d["store_sems"].at[q])

        barrier = pltpu.get_barrier_semaphore()
        for nbr in (left, right):
            pl.semaphore_signal(barrier, inc=1, device_id=(nbr,),
                                device_id_type=MESH)
        pl.semaphore_wait(barrier, 2)

        for d in dirs:
            cp = pltpu.make_async_copy(
                p_ref.at[pl.ds(d["base"] + my * CHUNK, CHUNK), :],
                d["local"].at[0], d["copy_sems"].at[0, 0])
            cp.start()
            cp.wait()
            d["send"][0, :, :] = d["local"][0, :, :].astype(jnp.bfloat16)
        for d in dirs:
            for j in range(SUBS):
                rdma(d, 0, j).start()
                load(d, 0, j).start()

        pending_stores = {0: [], 1: []}
        for t in range(N_STEPS):
            p = t % 2
            np_ = (t + 1) % 2
            rs = t < N_DEV - 1

            if t + 1 < N_DEV - 1:
                for d in dirs:
                    for j in range(SUBS):
                        load(d, t + 1, j).start()

            for st in pending_stores[np_]:
                st.wait()
            pending_stores[np_] = []

            for j in range(SUBS):
                for d in dirs:
                    rdma(d, t, j).wait_recv()
                    if t >= 1:
                        rdma(d, t - 1, j).wait_send()
                    sl = slice(j * SUB, (j + 1) * SUB)
                    if t < N_DEV - 2:
                        load(d, t, j).wait()
                        d["send"][np_, sl, :] = (
                            d["recv"][p, sl, :].astype(jnp.float32)
                            + d["local"][p, sl, :]).astype(jnp.bfloat16)
                    elif t == N_DEV - 2:
                        load(d, t, j).wait()
                        d["fst"][np_, sl, :] = (
                            d["recv"][p, sl, :].astype(jnp.float32)
                            + d["local"][p, sl, :])
                        d["send"][np_, sl, :] = (
                            d["fst"][np_, sl, :].astype(jnp.bfloat16))
                    else:
                        d["send"][np_, sl, :] = d["recv"][p, sl, :]
                        d["fst"][np_, sl, :] = (
                            d["recv"][p, sl, :].astype(jnp.float32))
                    if t <= N_STEPS - 3:
                        pl.semaphore_signal(d["credit"], inc=1,
                                            device_id=(d["ack"],),
                                            device_id_type=MESH)
                    if t + 1 < N_STEPS:
                        if t + 1 >= 2:
                            pl.semaphore_wait(d["credit"], 1)
                        rdma(d, t + 1, j).start()

            if t >= N_DEV - 2:
                for d in dirs:
                    st = store(d, t)
                    st.start()
                    pending_stores[np_].append(st)

        for d in dirs:
            for j in range(SUBS):
                rdma(d, N_STEPS - 1, j).wait_send()
        for q in (0, 1):
            for st in pending_stores[q]:
                st.wait()

    return pl.pallas_call(
        body,
        out_shape=jax.ShapeDtypeStruct((M, N), jnp.float32),
        in_specs=[pl.BlockSpec(memory_space=pl.ANY)],
        out_specs=pl.BlockSpec(memory_space=pl.ANY),
        scratch_shapes=[
            pltpu.VMEM((2, CHUNK, N), jnp.bfloat16),
            pltpu.VMEM((2, CHUNK, N), jnp.bfloat16),
            pltpu.VMEM((2, CHUNK, N), jnp.float32),
            pltpu.VMEM((2, CHUNK, N), jnp.float32),
            pltpu.VMEM((2, CHUNK, N), jnp.bfloat16),
            pltpu.VMEM((2, CHUNK, N), jnp.bfloat16),
            pltpu.VMEM((2, CHUNK, N), jnp.float32),
            pltpu.VMEM((2, CHUNK, N), jnp.float32),
            pltpu.SemaphoreType.DMA((2, SUBS)),
            pltpu.SemaphoreType.DMA((2, SUBS)),
            pltpu.SemaphoreType.DMA((2, SUBS)),
            pltpu.SemaphoreType.DMA((2,)),
            pltpu.SemaphoreType.DMA((2, SUBS)),
            pltpu.SemaphoreType.DMA((2, SUBS)),
            pltpu.SemaphoreType.DMA((2, SUBS)),
            pltpu.SemaphoreType.DMA((2,)),
            pltpu.SemaphoreType.REGULAR,
            pltpu.SemaphoreType.REGULAR,
        ],
        compiler_params=pltpu.CompilerParams(
            collective_id=0, vmem_limit_bytes=80 * 1024 * 1024),
    )(partial)


def kernel(x, w_mat, scale_x, scale_w):
    s = scale_x[0].astype(jnp.float32) * scale_w[0].astype(jnp.float32)
    partial = lax.dot_general(
        x, w_mat, (((1,), (0,)), ((), ())),
        preferred_element_type=jnp.float32)
    partial = partial * s
    return _allreduce(partial)


# device time: 807980 ns/iter; 1.8567x vs baseline; 1.0215x over previous
import jax
import jax.numpy as jnp
from jax import lax
from jax.experimental import pallas as pl
from jax.experimental.pallas import tpu as pltpu

N_DEV = 16
M = 4096
N = 8192
HALF = M // 2
CHUNK = HALF // N_DEV
SUBS = 2
SUB = CHUNK // SUBS
N_STEPS = 2 * (N_DEV - 1)
MESH = pl.DeviceIdType.MESH


def _allreduce(partial):
    def body(p_ref, out_ref,
             send_r, recv_r, local_r, fst_r, send_l, recv_l, local_l, fst_l,
             send_sems_r, recv_sems_r, copy_sems_r, store_sems_r,
             send_sems_l, recv_sems_l, copy_sems_l, store_sems_l,
             credit_r, credit_l):
        my = lax.axis_index("i")
        left = jnp.mod(my - 1, N_DEV)
        right = jnp.mod(my + 1, N_DEV)

        dirs = (
            dict(peer=right, ack=left, base=0, send=send_r, recv=recv_r,
                 local=local_r, fst=fst_r, send_sems=send_sems_r,
                 recv_sems=recv_sems_r, copy_sems=copy_sems_r,
                 store_sems=store_sems_r, credit=credit_r, sign=-1),
            dict(peer=left, ack=right, base=HALF, send=send_l, recv=recv_l,
                 local=local_l, fst=fst_l, send_sems=send_sems_l,
                 recv_sems=recv_sems_l, copy_sems=copy_sems_l,
                 store_sems=store_sems_l, credit=credit_l, sign=+1),
        )

        def rdma(d, t, j):
            p = t % 2
            return pltpu.make_async_remote_copy(
                src_ref=d["send"].at[p, pl.ds(j * SUB, SUB), :],
                dst_ref=d["recv"].at[p, pl.ds(j * SUB, SUB), :],
                send_sem=d["send_sems"].at[p, j],
                recv_sem=d["recv_sems"].at[p, j],
                device_id=(d["peer"],), device_id_type=MESH)

        def in_idx(d, t):
            return jnp.mod(my + d["sign"] * (t + 1), N_DEV)

        def load(d, t, j):
            p = t % 2
            rows = in_idx(d, t) * CHUNK + j * SUB
            return pltpu.make_async_copy(
                p_ref.at[pl.ds(d["base"] + rows, SUB), :],
                d["local"].at[p, pl.ds(j * SUB, SUB), :],
                d["copy_sems"].at[p, j])

        def store(d, t):
            q = (t + 1) % 2
            if t == N_DEV - 2:
                idx = jnp.mod(my - d["sign"], N_DEV)
            else:
                idx = jnp.mod(my + d["sign"] * (t - (N_DEV - 1)), N_DEV)
            return pltpu.make_async_copy(
                d["fst"].at[q],
                out_ref.at[pl.ds(d["base"] + idx * CHUNK, CHUNK), :],
                d["store_sems"].at[q])

        barrier = pltpu.get_barrier_semaphore()
        for nbr in (left, right):
            pl.semaphore_signal(barrier, inc=1, device_id=(nbr,),
                                device_id_type=MESH)
        pl.semaphore_wait(barrier, 2)

        for d in dirs:
            cp = pltpu.make_async_copy(
                p_ref.at[pl.ds(d["base"] + my * CHUNK, CHUNK), :],
                d["send"].at[0], d["copy_sems"].at[0, 0])
            cp.start()
            cp.wait()
        for d in dirs:
            for j in range(SUBS):
                rdma(d, 0, j).start()
                load(d, 0, j).start()

        pending_stores = {0: [], 1: []}
        for t in range(N_STEPS):
            p = t % 2
            np_ = (t + 1) % 2
            rs = t < N_DEV - 1

            if t + 1 < N_DEV - 1:
                for d in dirs:
                    for j in range(SUBS):
                        load(d, t + 1, j).start()

            for st in pending_stores[np_]:
                st.wait()
            pending_stores[np_] = []

            for j in range(SUBS):
                for d in dirs:
                    rdma(d, t, j).wait_recv()
                    if t >= 1:
                        rdma(d, t - 1, j).wait_send()
                    sl = slice(j * SUB, (j + 1) * SUB)
                    if rs:
                        load(d, t, j).wait()
                        d["send"][np_, sl, :] = (
                            d["recv"][p, sl, :] + d["local"][p, sl, :])
                    else:
                        d["send"][np_, sl, :] = d["recv"][p, sl, :]
                    if t + 1 < N_STEPS:
                        if t + 1 >= 2:
                            pl.semaphore_wait(d["credit"], 1)
                        rdma(d, t + 1, j).start()
                    if t >= N_DEV - 2:
                        d["fst"][np_, sl, :] = (
                            d["send"][np_, sl, :].astype(jnp.float32))
                    if t <= N_STEPS - 3:
                        pl.semaphore_signal(d["credit"], inc=1,
                                            device_id=(d["ack"],),
                                            device_id_type=MESH)

            if t >= N_DEV - 2:
                for d in dirs:
                    st = store(d, t)
                    st.start()
                    pending_stores[np_].append(st)

        for d in dirs:
            for j in range(SUBS):
                rdma(d, N_STEPS - 1, j).wait_send()
        for q in (0, 1):
            for st in pending_stores[q]:
                st.wait()

    return pl.pallas_call(
        body,
        out_shape=jax.ShapeDtypeStruct((M, N), jnp.float32),
        in_specs=[pl.BlockSpec(memory_space=pl.ANY)],
        out_specs=pl.BlockSpec(memory_space=pl.ANY),
        scratch_shapes=[
            pltpu.VMEM((2, CHUNK, N), jnp.bfloat16),
            pltpu.VMEM((2, CHUNK, N), jnp.bfloat16),
            pltpu.VMEM((2, CHUNK, N), jnp.bfloat16),
            pltpu.VMEM((2, CHUNK, N), jnp.float32),
            pltpu.VMEM((2, CHUNK, N), jnp.bfloat16),
            pltpu.VMEM((2, CHUNK, N), jnp.bfloat16),
            pltpu.VMEM((2, CHUNK, N), jnp.bfloat16),
            pltpu.VMEM((2, CHUNK, N), jnp.float32),
            pltpu.SemaphoreType.DMA((2, SUBS)),
            pltpu.SemaphoreType.DMA((2, SUBS)),
            pltpu.SemaphoreType.DMA((2, SUBS)),
            pltpu.SemaphoreType.DMA((2,)),
            pltpu.SemaphoreType.DMA((2, SUBS)),
            pltpu.SemaphoreType.DMA((2, SUBS)),
            pltpu.SemaphoreType.DMA((2, SUBS)),
            pltpu.SemaphoreType.DMA((2,)),
            pltpu.SemaphoreType.REGULAR,
            pltpu.SemaphoreType.REGULAR,
        ],
        compiler_params=pltpu.CompilerParams(
            collective_id=0, vmem_limit_bytes=80 * 1024 * 1024),
    )(partial)


def kernel(x, w_mat, scale_x, scale_w):
    s = scale_x[0].astype(jnp.float32) * scale_w[0].astype(jnp.float32)
    partial = lax.dot_general(
        x, w_mat, (((1,), (0,)), ((), ())),
        preferred_element_type=jnp.float32)
    partial = (partial * s).astype(jnp.bfloat16)
    return _allreduce(partial)


# device time: 806668 ns/iter; 1.8597x vs baseline; 1.0016x over previous
import jax
import jax.numpy as jnp
from jax import lax
from jax.experimental import pallas as pl
from jax.experimental.pallas import tpu as pltpu

N_DEV = 16
M = 4096
N = 8192
HALF = M // 2
CHUNK = HALF // N_DEV
SUBS = 4
SUB = CHUNK // SUBS
N_STEPS = 2 * (N_DEV - 1)
MESH = pl.DeviceIdType.MESH


def _allreduce(partial):
    def body(p_ref, out_ref,
             send_r, recv_r, local_r, fst_r, send_l, recv_l, local_l, fst_l,
             send_sems_r, recv_sems_r, copy_sems_r, store_sems_r,
             send_sems_l, recv_sems_l, copy_sems_l, store_sems_l,
             credit_r, credit_l):
        my = lax.axis_index("i")
        left = jnp.mod(my - 1, N_DEV)
        right = jnp.mod(my + 1, N_DEV)

        dirs = (
            dict(peer=right, ack=left, base=0, send=send_r, recv=recv_r,
                 local=local_r, fst=fst_r, send_sems=send_sems_r,
                 recv_sems=recv_sems_r, copy_sems=copy_sems_r,
                 store_sems=store_sems_r, credit=credit_r, sign=-1),
            dict(peer=left, ack=right, base=HALF, send=send_l, recv=recv_l,
                 local=local_l, fst=fst_l, send_sems=send_sems_l,
                 recv_sems=recv_sems_l, copy_sems=copy_sems_l,
                 store_sems=store_sems_l, credit=credit_l, sign=+1),
        )

        def rdma(d, t, j):
            p = t % 2
            return pltpu.make_async_remote_copy(
                src_ref=d["send"].at[p, pl.ds(j * SUB, SUB), :],
                dst_ref=d["recv"].at[p, pl.ds(j * SUB, SUB), :],
                send_sem=d["send_sems"].at[p, j],
                recv_sem=d["recv_sems"].at[p, j],
                device_id=(d["peer"],), device_id_type=MESH)

        def in_idx(d, t):
            return jnp.mod(my + d["sign"] * (t + 1), N_DEV)

        def load(d, t, j):
            p = t % 2
            rows = in_idx(d, t) * CHUNK + j * SUB
            return pltpu.make_async_copy(
                p_ref.at[pl.ds(d["base"] + rows, SUB), :],
                d["local"].at[p, pl.ds(j * SUB, SUB), :],
                d["copy_sems"].at[p, j])

        def store(d, t):
            q = (t + 1) % 2
            if t == N_DEV - 2:
                idx = jnp.mod(my - d["sign"], N_DEV)
            else:
                idx = jnp.mod(my + d["sign"] * (t - (N_DEV - 1)), N_DEV)
            return pltpu.make_async_copy(
                d["fst"].at[q],
                out_ref.at[pl.ds(d["base"] + idx * CHUNK, CHUNK), :],
                d["store_sems"].at[q])

        barrier = pltpu.get_barrier_semaphore()
        for nbr in (left, right):
            pl.semaphore_signal(barrier, inc=1, device_id=(nbr,),
                                device_id_type=MESH)
        pl.semaphore_wait(barrier, 2)

        for d in dirs:
            cp = pltpu.make_async_copy(
                p_ref.at[pl.ds(d["base"] + my * CHUNK, CHUNK), :],
                d["send"].at[0], d["copy_sems"].at[0, 0])
            cp.start()
            cp.wait()
        for d in dirs:
            for j in range(SUBS):
                rdma(d, 0, j).start()
                load(d, 0, j).start()

        pending_stores = {0: [], 1: []}
        for t in range(N_STEPS):
            p = t % 2
            np_ = (t + 1) % 2
            rs = t < N_DEV - 1

            if t + 1 < N_DEV - 1:
                for d in dirs:
                    for j in range(SUBS):
                        load(d, t + 1, j).start()

            for st in pending_stores[np_]:
                st.wait()
            pending_stores[np_] = []

            for j in range(SUBS):
                for d in dirs:
                    rdma(d, t, j).wait_recv()
                    if t >= 1:
                        rdma(d, t - 1, j).wait_send()
                    sl = slice(j * SUB, (j + 1) * SUB)
                    if rs:
                        load(d, t, j).wait()
                        d["send"][np_, sl, :] = (
                            d["recv"][p, sl, :] + d["local"][p, sl, :])
                    else:
                        d["send"][np_, sl, :] = d["recv"][p, sl, :]
                    if t + 1 < N_STEPS:
                        if t + 1 >= 2:
                            pl.semaphore_wait(d["credit"], 1)
                        rdma(d, t + 1, j).start()
                    if t >= N_DEV - 2:
                        d["fst"][np_, sl, :] = (
                            d["send"][np_, sl, :].astype(jnp.float32))
                    if t <= N_STEPS - 3:
                        pl.semaphore_signal(d["credit"], inc=1,
                                            device_id=(d["ack"],),
                                            device_id_type=MESH)

            if t >= N_DEV - 2:
                for d in dirs:
                    st = store(d, t)
                    st.start()
                    pending_stores[np_].append(st)

        for d in dirs:
            for j in range(SUBS):
                rdma(d, N_STEPS - 1, j).wait_send()
        for q in (0, 1):
            for st in pending_stores[q]:
                st.wait()

    return pl.pallas_call(
        body,
        out_shape=jax.ShapeDtypeStruct((M, N), jnp.float32),
        in_specs=[pl.BlockSpec(memory_space=pl.ANY)],
        out_specs=pl.BlockSpec(memory_space=pl.ANY),
        scratch_shapes=[
            pltpu.VMEM((2, CHUNK, N), jnp.bfloat16),
            pltpu.VMEM((2, CHUNK, N), jnp.bfloat16),
            pltpu.VMEM((2, CHUNK, N), jnp.bfloat16),
            pltpu.VMEM((2, CHUNK, N), jnp.float32),
            pltpu.VMEM((2, CHUNK, N), jnp.bfloat16),
            pltpu.VMEM((2, CHUNK, N), jnp.bfloat16),
            pltpu.VMEM((2, CHUNK, N), jnp.bfloat16),
            pltpu.VMEM((2, CHUNK, N), jnp.float32),
            pltpu.SemaphoreType.DMA((2, SUBS)),
            pltpu.SemaphoreType.DMA((2, SUBS)),
            pltpu.SemaphoreType.DMA((2, SUBS)),
            pltpu.SemaphoreType.DMA((2,)),
            pltpu.SemaphoreType.DMA((2, SUBS)),
            pltpu.SemaphoreType.DMA((2, SUBS)),
            pltpu.SemaphoreType.DMA((2, SUBS)),
            pltpu.SemaphoreType.DMA((2,)),
            pltpu.SemaphoreType.REGULAR,
            pltpu.SemaphoreType.REGULAR,
        ],
        compiler_params=pltpu.CompilerParams(
            collective_id=0, vmem_limit_bytes=80 * 1024 * 1024),
    )(partial)


def kernel(x, w_mat, scale_x, scale_w):
    s = scale_x[0].astype(jnp.float32) * scale_w[0].astype(jnp.float32)
    partial = lax.dot_general(
        x, w_mat, (((1,), (0,)), ((), ())),
        preferred_element_type=jnp.float32)
    partial = (partial * s).astype(jnp.bfloat16)
    return _allreduce(partial)
